# baseline (speedup 1.0000x reference)
import functools

import jax
import jax.numpy as jnp
from jax import lax
from jax.experimental import pallas as pl
from jax.experimental.pallas import tpu as pltpu
from jax.experimental.pallas import tpu_sc as plsc


def _make_queue_kernel(B, D, dtype):
    info = plsc.get_sparse_core_info()
    nw = info.num_cores * info.num_subcores
    b_per_w = B // nw
    assert b_per_w * nw == B

    mesh = plsc.VectorSubcoreMesh(core_axis_name="c", subcore_axis_name="s")

    @functools.partial(
        pl.kernel,
        out_type=jax.ShapeDtypeStruct((B, D), dtype),
        mesh=mesh,
        scratch_types=[
            pltpu.SemaphoreType.DMA,
        ],
    )
    def queue_kernel(item_hbm, out_hbm, sem):
        wid = lax.axis_index("s") * info.num_cores + lax.axis_index("c")
        base = wid * b_per_w
        pltpu.async_copy(
            item_hbm.at[pl.ds(base, b_per_w)],
            out_hbm.at[pl.ds(base, b_per_w)],
            sem,
        ).wait()

    return queue_kernel


def kernel(item, out, order_indices):
    B, D = item.shape
    return _make_queue_kernel(B, D, item.dtype)(item)


# speedup vs baseline: 10.4330x; 10.4330x over previous
import functools

import jax
import jax.numpy as jnp
from jax import lax
from jax.experimental import pallas as pl
from jax.experimental.pallas import tpu as pltpu
from jax.experimental.pallas import tpu_sc as plsc


def _make_queue_kernel(B, D, dtype):
    info = plsc.get_sparse_core_info()
    nw = info.num_cores * info.num_subcores
    b_per_w = B // nw
    assert b_per_w * nw == B
    nchunks = 4
    chunk = b_per_w // nchunks
    assert chunk * nchunks == b_per_w

    mesh = plsc.VectorSubcoreMesh(core_axis_name="c", subcore_axis_name="s")

    @functools.partial(
        pl.kernel,
        out_type=jax.ShapeDtypeStruct((B, D), dtype),
        mesh=mesh,
        scratch_types=[
            pltpu.VMEM((2, chunk, D), dtype),
            pltpu.SemaphoreType.DMA,
            pltpu.SemaphoreType.DMA,
            pltpu.SemaphoreType.DMA,
            pltpu.SemaphoreType.DMA,
        ],
    )
    def queue_kernel(item_hbm, out_hbm, bufs, si0, si1, so0, so1):
        wid = lax.axis_index("s") * info.num_cores + lax.axis_index("c")
        base = wid * b_per_w
        sin = (si0, si1)
        sout = (so0, so1)

        def in_copy(i):
            return pltpu.async_copy(
                item_hbm.at[pl.ds(base + i * chunk, chunk)],
                bufs.at[i % 2],
                sin[i % 2],
            )

        def out_copy(i):
            return pltpu.async_copy(
                bufs.at[i % 2],
                out_hbm.at[pl.ds(base + i * chunk, chunk)],
                sout[i % 2],
            )

        in_h, out_h = {}, {}
        in_h[0] = in_copy(0)
        for i in range(nchunks):
            if i + 1 < nchunks:
                if i >= 1:
                    out_h[i - 1].wait()
                in_h[i + 1] = in_copy(i + 1)
            in_h[i].wait()
            out_h[i] = out_copy(i)
        out_h[nchunks - 2].wait()
        out_h[nchunks - 1].wait()

    return queue_kernel


def kernel(item, out, order_indices):
    B, D = item.shape
    return _make_queue_kernel(B, D, item.dtype)(item)
